# Initial kernel scaffold; baseline (speedup 1.0000x reference)
import jax
import jax.numpy as jnp
from jax.experimental import pallas as pl


def kernel(nfeats, g, dis, params):
    raise NotImplementedError("write your pallas kernel here")


# speedup vs baseline: 101.7082x; 101.7082x over previous
import functools

import jax
import jax.numpy as jnp
from jax import lax
from jax.experimental import pallas as pl
from jax.experimental.pallas import tpu as pltpu
from jax.experimental.pallas import tpu_sc as plsc

N = 1024
E = 65536
H = 6
D = 64
HID = H * D

_ROWS = N // 16
_EHALF = E // 2
_CHUNK = 8192
_RT = 256
_F32 = jnp.float32


def _count_body(g_hbm, zeros_hbm, out_hbm, cmat, srcb, dstb):
    c = lax.axis_index("c")
    s = lax.axis_index("s")
    base = s * _ROWS
    pltpu.sync_copy(zeros_hbm, cmat)
    e0 = c * _EHALF
    ones = jnp.ones((16,), _F32)

    def chunk(ci, carry):
        off = e0 + ci * _CHUNK
        pltpu.sync_copy(g_hbm.at[0, pl.ds(off, _CHUNK)], srcb)
        pltpu.sync_copy(g_hbm.at[1, pl.ds(off, _CHUNK)], dstb)

        def step(j, carry2):
            d = dstb[pl.ds(j * 16, 16)]
            sv = srcb[pl.ds(j * 16, 16)]
            rel = d - base
            m = (rel >= 0) & (rel < _ROWS)
            relc = jnp.where(m, rel, 0)
            flat = relc * N + sv
            plsc.addupdate_scatter(cmat, [flat], ones, mask=m)
            return carry2

        return lax.fori_loop(0, _CHUNK // 16, step, carry)

    lax.fori_loop(0, _EHALF // _CHUNK, chunk, 0)
    pltpu.sync_copy(cmat, out_hbm.at[c, s])


def _build_counts(g, zeros):
    call = pl.kernel(
        _count_body,
        out_type=jax.ShapeDtypeStruct((2, 16, _ROWS * N), _F32),
        mesh=plsc.VectorSubcoreMesh(core_axis_name="c", subcore_axis_name="s"),
        compiler_params=pltpu.CompilerParams(needs_layout_passes=False),
        scratch_types=[
            pltpu.VMEM((_ROWS * N,), _F32),
            pltpu.VMEM((_CHUNK,), jnp.int32),
            pltpu.VMEM((_CHUNK,), jnp.int32),
        ],
    )
    return call(g, zeros).reshape(2, N, N)


def _fc_body(x_ref, w_ref, al_ref, ar_ref, z_ref, el_ref, er_ref):
    z = jnp.dot(x_ref[...], w_ref[...], preferred_element_type=_F32,
                precision=lax.Precision.HIGHEST)
    z_ref[...] = z
    els, ers = [], []
    for h in range(H):
        zh = z[:, h * D:(h + 1) * D]
        els.append(jnp.sum(zh * al_ref[:, h * D:(h + 1) * D], axis=1, keepdims=True))
        ers.append(jnp.sum(zh * ar_ref[:, h * D:(h + 1) * D], axis=1, keepdims=True))
    el_ref[...] = jnp.concatenate(els, axis=1)
    er_ref[...] = jnp.concatenate(ers, axis=1)


def _fc_call(x, w, al, ar):
    k = x.shape[1]
    return pl.pallas_call(
        _fc_body,
        grid=(N // _RT,),
        in_specs=[
            pl.BlockSpec((_RT, k), lambda i: (i, 0)),
            pl.BlockSpec((k, HID), lambda i: (0, 0)),
            pl.BlockSpec((1, HID), lambda i: (0, 0)),
            pl.BlockSpec((1, HID), lambda i: (0, 0)),
        ],
        out_specs=[
            pl.BlockSpec((_RT, HID), lambda i: (i, 0)),
            pl.BlockSpec((_RT, H), lambda i: (i, 0)),
            pl.BlockSpec((_RT, H), lambda i: (i, 0)),
        ],
        out_shape=[
            jax.ShapeDtypeStruct((N, HID), _F32),
            jax.ShapeDtypeStruct((N, H), _F32),
            jax.ShapeDtypeStruct((N, H), _F32),
        ],
    )(x, w, al, ar)


def _attn_body(c0_ref, c1_ref, elt_ref, er_ref, feat_ref, b_ref, out_ref, *, last):
    cm = c0_ref[...] + c1_ref[...]
    pos = cm > 0.0
    acc = None
    outs = []
    for h in range(H):
        el_h = elt_ref[h]
        t = er_ref[:, h:h + 1] + el_h[None, :]
        e = jnp.maximum(t, 0.2 * t)
        em = jnp.max(jnp.where(pos, e, -1e30), axis=1, keepdims=True)
        p = cm * jnp.exp(jnp.minimum(e - em, 0.0))
        dn = jnp.sum(p, axis=1, keepdims=True)
        a = p * (1.0 / (dn + 1e-12))
        oh = jnp.dot(a, feat_ref[:, h * D:(h + 1) * D],
                     preferred_element_type=_F32,
                     precision=lax.Precision.HIGHEST)
        if last:
            acc = oh if acc is None else acc + oh
        else:
            outs.append(oh)
    if last:
        out_ref[...] = acc * (1.0 / H) + b_ref[...]
    else:
        o = jnp.concatenate(outs, axis=1) + b_ref[...]
        out_ref[...] = jnp.where(o > 0.0, o, jnp.exp(jnp.minimum(o, 0.0)) - 1.0)


def _attn_call(c0, c1, elt, er, feat, b, last):
    od = D if last else HID
    body = functools.partial(_attn_body, last=last)
    return pl.pallas_call(
        body,
        grid=(N // _RT,),
        in_specs=[
            pl.BlockSpec((_RT, N), lambda i: (i, 0)),
            pl.BlockSpec((_RT, N), lambda i: (i, 0)),
            pl.BlockSpec((H, N), lambda i: (0, 0)),
            pl.BlockSpec((_RT, H), lambda i: (i, 0)),
            pl.BlockSpec((N, HID), lambda i: (0, 0)),
            pl.BlockSpec((1, od), lambda i: (0, 0)),
        ],
        out_specs=pl.BlockSpec((_RT, od), lambda i: (i, 0)),
        out_shape=jax.ShapeDtypeStruct((N, od), _F32),
    )(c0, c1, elt, er, feat, b)


def _od_body(dis_ref, emb_ref, embt_ref, wp1_ref, wp2_ref, sc_ref, od_ref):
    lin1 = jnp.dot(wp1_ref[...], embt_ref[...], preferred_element_type=_F32,
                   precision=lax.Precision.HIGHEST)
    lin2 = jnp.sum(emb_ref[...] * wp2_ref[...], axis=1, keepdims=True)
    od_ref[...] = jnp.tanh(lin2 + lin1 + dis_ref[...] * sc_ref[:, 0:1]
                           + sc_ref[:, 1:2])


def _od_call(dis, emb, embt, wp1, wp2, sc):
    return pl.pallas_call(
        _od_body,
        grid=(N // _RT,),
        in_specs=[
            pl.BlockSpec((_RT, N), lambda i: (i, 0)),
            pl.BlockSpec((_RT, D), lambda i: (i, 0)),
            pl.BlockSpec((D, N), lambda i: (0, 0)),
            pl.BlockSpec((1, D), lambda i: (0, 0)),
            pl.BlockSpec((1, D), lambda i: (0, 0)),
            pl.BlockSpec((1, 2), lambda i: (0, 0)),
        ],
        out_specs=pl.BlockSpec((_RT, N), lambda i: (i, 0)),
        out_shape=jax.ShapeDtypeStruct((N, N), _F32),
    )(dis, emb, embt, wp1, wp2, sc)


def kernel(nfeats, g, dis, params):
    cparts = _build_counts(g, jnp.zeros((_ROWS * N,), _F32))
    c0, c1 = cparts[0], cparts[1]

    h = jnp.pad(nfeats, ((0, 0), (0, 256 - nfeats.shape[1])))
    for l in range(4):
        w = params[f"W{l}"]
        if l == 0:
            w = jnp.pad(w, ((0, 256 - w.shape[0]), (0, 0)))
        al = params[f"al{l}"].reshape(1, HID)
        ar = params[f"ar{l}"].reshape(1, HID)
        z, el, er = _fc_call(h, w, al, ar)
        last = l == 3
        b = params[f"b{l}"]
        bb = b.reshape(H, D).mean(axis=0)[None, :] if last else b[None, :]
        h = _attn_call(c0, c1, el.T, er, z, bb, last)

    emb = h
    wp = params["Wp"][:, 0]
    sc = jnp.stack([wp[128], params["bp"][0]]).reshape(1, 2)
    return _od_call(dis, emb, emb.T, wp[:64][None, :], wp[64:128][None, :], sc)


# speedup vs baseline: 134.6628x; 1.3240x over previous
import functools

import jax
import jax.numpy as jnp
from jax import lax
from jax.experimental import pallas as pl
from jax.experimental.pallas import tpu as pltpu
from jax.experimental.pallas import tpu_sc as plsc

N = 1024
E = 65536
H = 6
D = 64
HID = H * D

_ROWS = N // 16
_EHALF = E // 2
_CHUNK = 8192
_RT = 256
_F32 = jnp.float32
_PREC = lax.Precision.HIGHEST
_PREC_AGG = lax.Precision.DEFAULT


def _count_body(g_hbm, zeros_hbm, out_hbm, cmat, srcb, dstb):
    c = lax.axis_index("c")
    s = lax.axis_index("s")
    base = s * _ROWS
    pltpu.sync_copy(zeros_hbm, cmat)
    e0 = c * _EHALF
    ones = jnp.ones((16,), _F32)

    def chunk(ci, carry):
        off = e0 + ci * _CHUNK
        pltpu.sync_copy(g_hbm.at[0, pl.ds(off, _CHUNK)], srcb)
        pltpu.sync_copy(g_hbm.at[1, pl.ds(off, _CHUNK)], dstb)

        def step(j, carry2):
            for u in range(4):
                d = dstb[pl.ds(j * 64 + u * 16, 16)]
                sv = srcb[pl.ds(j * 64 + u * 16, 16)]
                rel = d - base
                m = (rel >= 0) & (rel < _ROWS)
                relc = jnp.where(m, rel, 0)
                flat = relc * N + sv
                plsc.addupdate_scatter(cmat, [flat], ones, mask=m)
            return carry2

        return lax.fori_loop(0, _CHUNK // 64, step, carry)

    lax.fori_loop(0, _EHALF // _CHUNK, chunk, 0)
    pltpu.sync_copy(cmat, out_hbm.at[c, s])


def _build_counts(g, zeros):
    call = pl.kernel(
        _count_body,
        out_type=jax.ShapeDtypeStruct((2, 16, _ROWS * N), _F32),
        mesh=plsc.VectorSubcoreMesh(core_axis_name="c", subcore_axis_name="s"),
        compiler_params=pltpu.CompilerParams(needs_layout_passes=False),
        scratch_types=[
            pltpu.VMEM((_ROWS * N,), _F32),
            pltpu.VMEM((_CHUNK,), jnp.int32),
            pltpu.VMEM((_CHUNK,), jnp.int32),
        ],
    )
    return call(g, zeros).reshape(2, N, N)


def _fc_body(x_ref, w_ref, al_ref, ar_ref, z_ref, el_ref, er_ref):
    z = jnp.dot(x_ref[...], w_ref[...], preferred_element_type=_F32,
                precision=_PREC)
    z_ref[...] = z
    els, ers = [], []
    for h in range(H):
        zh = z[:, h * D:(h + 1) * D]
        els.append(jnp.sum(zh * al_ref[:, h * D:(h + 1) * D], axis=1, keepdims=True))
        ers.append(jnp.sum(zh * ar_ref[:, h * D:(h + 1) * D], axis=1, keepdims=True))
    el_ref[...] = jnp.concatenate(els, axis=1)
    er_ref[...] = jnp.concatenate(ers, axis=1)


def _fc_call(x, w, al, ar):
    k = x.shape[1]
    return pl.pallas_call(
        _fc_body,
        grid=(N // _RT,),
        in_specs=[
            pl.BlockSpec((_RT, k), lambda i: (i, 0)),
            pl.BlockSpec((k, HID), lambda i: (0, 0)),
            pl.BlockSpec((1, HID), lambda i: (0, 0)),
            pl.BlockSpec((1, HID), lambda i: (0, 0)),
        ],
        out_specs=[
            pl.BlockSpec((_RT, HID), lambda i: (i, 0)),
            pl.BlockSpec((_RT, H), lambda i: (i, 0)),
            pl.BlockSpec((_RT, H), lambda i: (i, 0)),
        ],
        out_shape=[
            jax.ShapeDtypeStruct((N, HID), _F32),
            jax.ShapeDtypeStruct((N, H), _F32),
            jax.ShapeDtypeStruct((N, H), _F32),
        ],
    )(x, w, al, ar)


def _attn_body(*refs, first, last):
    if first:
        c0_ref, c1_ref, elt_ref, er_ref, feat_ref, b_ref, out_ref, cm_ref = refs
        cm = c0_ref[...] + c1_ref[...]
        cm_ref[...] = cm
    else:
        cin_ref, elt_ref, er_ref, feat_ref, b_ref, out_ref = refs
        cm = cin_ref[...]
    pos = cm > 0.0
    acc = None
    outs = []
    for h in range(H):
        el_h = elt_ref[h]
        t = er_ref[:, h:h + 1] + el_h[None, :]
        e = jnp.maximum(t, 0.2 * t)
        em = jnp.max(jnp.where(pos, e, -1e30), axis=1, keepdims=True)
        p = cm * jnp.exp(jnp.minimum(e - em, 0.0))
        dn = jnp.sum(p, axis=1, keepdims=True)
        oh = jnp.dot(p, feat_ref[:, h * D:(h + 1) * D],
                     preferred_element_type=_F32, precision=_PREC_AGG)
        oh = oh * (1.0 / (dn + 1e-12))
        if last:
            acc = oh if acc is None else acc + oh
        else:
            outs.append(oh)
    if last:
        out_ref[...] = acc * (1.0 / H) + b_ref[...]
    else:
        o = jnp.concatenate(outs, axis=1) + b_ref[...]
        out_ref[...] = jnp.where(o > 0.0, o, jnp.exp(jnp.minimum(o, 0.0)) - 1.0)


def _attn_call(cparts, elt, er, feat, b, first, last):
    od = D if last else HID
    body = functools.partial(_attn_body, first=first, last=last)
    cspecs = [pl.BlockSpec((_RT, N), lambda i: (i, 0))] * (2 if first else 1)
    out_specs = pl.BlockSpec((_RT, od), lambda i: (i, 0))
    out_shape = jax.ShapeDtypeStruct((N, od), _F32)
    if first:
        out_specs = [out_specs, pl.BlockSpec((_RT, N), lambda i: (i, 0))]
        out_shape = [out_shape, jax.ShapeDtypeStruct((N, N), _F32)]
    return pl.pallas_call(
        body,
        grid=(N // _RT,),
        in_specs=cspecs + [
            pl.BlockSpec((H, N), lambda i: (0, 0)),
            pl.BlockSpec((_RT, H), lambda i: (i, 0)),
            pl.BlockSpec((N, HID), lambda i: (0, 0)),
            pl.BlockSpec((1, od), lambda i: (0, 0)),
        ],
        out_specs=out_specs,
        out_shape=out_shape,
    )(*cparts, elt, er, feat, b)


def _od_body(dis_ref, emb_ref, embt_ref, wp1_ref, wp2_ref, sc_ref, od_ref):
    lin1 = jnp.dot(wp1_ref[...], embt_ref[...], preferred_element_type=_F32,
                   precision=_PREC)
    lin2 = jnp.sum(emb_ref[...] * wp2_ref[...], axis=1, keepdims=True)
    od_ref[...] = jnp.tanh(lin2 + lin1 + dis_ref[...] * sc_ref[:, 0:1]
                           + sc_ref[:, 1:2])


def _od_call(dis, emb, embt, wp1, wp2, sc):
    return pl.pallas_call(
        _od_body,
        grid=(N // _RT,),
        in_specs=[
            pl.BlockSpec((_RT, N), lambda i: (i, 0)),
            pl.BlockSpec((_RT, D), lambda i: (i, 0)),
            pl.BlockSpec((D, N), lambda i: (0, 0)),
            pl.BlockSpec((1, D), lambda i: (0, 0)),
            pl.BlockSpec((1, D), lambda i: (0, 0)),
            pl.BlockSpec((1, 2), lambda i: (0, 0)),
        ],
        out_specs=pl.BlockSpec((_RT, N), lambda i: (i, 0)),
        out_shape=jax.ShapeDtypeStruct((N, N), _F32),
    )(dis, emb, embt, wp1, wp2, sc)


def kernel(nfeats, g, dis, params):
    cparts = _build_counts(g, jnp.zeros((_ROWS * N,), _F32))

    h = jnp.pad(nfeats, ((0, 0), (0, 256 - nfeats.shape[1])))
    cm = None
    for l in range(4):
        w = params[f"W{l}"]
        if l == 0:
            w = jnp.pad(w, ((0, 256 - w.shape[0]), (0, 0)))
        al = params[f"al{l}"].reshape(1, HID)
        ar = params[f"ar{l}"].reshape(1, HID)
        z, el, er = _fc_call(h, w, al, ar)
        first = l == 0
        last = l == 3
        b = params[f"b{l}"]
        bb = b.reshape(H, D).mean(axis=0)[None, :] if last else b[None, :]
        cin = [cparts[0], cparts[1]] if first else [cm]
        h = _attn_call(cin, el.T, er, z, bb, first, last)
        if first:
            h, cm = h

    emb = h
    wp = params["Wp"][:, 0]
    sc = jnp.stack([wp[128], params["bp"][0]]).reshape(1, 2)
    return _od_call(dis, emb, emb.T, wp[:64][None, :], wp[64:128][None, :], sc)
